# Initial kernel scaffold; baseline (speedup 1.0000x reference)
import jax
import jax.numpy as jnp
from jax.experimental import pallas as pl


def kernel(x, edge_index, edge_weight, W1, b1, W2, b2, W3, b3, W4, b4):
    raise NotImplementedError("write your pallas kernel here")


# speedup vs baseline: 9.3388x; 9.3388x over previous
import functools

import jax
import jax.numpy as jnp
from jax import lax
from jax.experimental import pallas as pl
from jax.experimental.pallas import tpu as pltpu
from jax.experimental.pallas import tpu_sc as plsc

N = 10000
NP = 10240
E = 320000
EP = 327680
BLK = 128
NBF = EP // BLK
NBE = EP // 2 // BLK
ROWS_PER_TILE = NP // 16


def _zero_vmem(buf, nrows, ncols):
  z = jnp.zeros((16,), jnp.float32)

  def row(i, _):
    for k in range(ncols // 16):
      buf[i, pl.ds(16 * k, 16)] = z
    return 0

  lax.fori_loop(0, nrows, row, 0)


def _make_deg_kernel():
  nbt = NBE // 16

  def body(sidx, ew, out, ibuf, ebuf, zbuf, deg_sp):
    c = lax.axis_index("c")
    s = lax.axis_index("s")
    def zrow(i, _):
      zbuf[pl.ds(16 * i, 16)] = jnp.zeros((16,), jnp.float32)
      return 0
    lax.fori_loop(0, ROWS_PER_TILE // 16, zrow, 0)
    pltpu.sync_copy(zbuf, deg_sp.at[pl.ds(s * ROWS_PER_TILE, ROWS_PER_TILE)])
    plsc.subcore_barrier()
    base = s * nbt
    pltpu.sync_copy(sidx.at[c, pl.ds(base, nbt)], ibuf)
    pltpu.sync_copy(ew.at[c, pl.ds(base, nbt)], ebuf)

    def blk(j, _):
      pltpu.sync_copy(ebuf.at[j], deg_sp.at[ibuf.at[j]], add=True)
      return 0

    lax.fori_loop(0, nbt, blk, 0)
    plsc.subcore_barrier()
    pltpu.sync_copy(deg_sp.at[pl.ds(s * ROWS_PER_TILE, ROWS_PER_TILE)],
                    out.at[c, pl.ds(s * ROWS_PER_TILE, ROWS_PER_TILE)])

  mesh = plsc.VectorSubcoreMesh(core_axis_name="c", subcore_axis_name="s")
  return pl.kernel(
      body,
      out_type=jax.ShapeDtypeStruct((2, NP), jnp.float32),
      mesh=mesh,
      scratch_types=[
          pltpu.VMEM((nbt, BLK), jnp.int32),
          pltpu.VMEM((nbt, BLK), jnp.float32),
          pltpu.VMEM((ROWS_PER_TILE,), jnp.float32),
          pltpu.VMEM_SHARED((NP,), jnp.float32),
      ],
  )


CH = 16


def _make_msg_kernel(nblocks_per_core, table_rows):
  nbt = nblocks_per_core // 16
  nchunks = nbt // CH

  def body(table, gidx, sidx, ew, out, gbuf, sbuf, ebuf, rowbuf, acc_sp):
    c = lax.axis_index("c")
    s = lax.axis_index("s")
    _zero_vmem(rowbuf, BLK, 128)
    for k in range(ROWS_PER_TILE // BLK):
      pltpu.sync_copy(
          rowbuf, acc_sp.at[pl.ds(s * ROWS_PER_TILE + k * BLK, BLK)])
    plsc.subcore_barrier()
    base = s * nbt

    def chunk(cc, _):
      pltpu.sync_copy(gidx.at[c, pl.ds(base + cc * CH, CH)], gbuf)
      pltpu.sync_copy(sidx.at[c, pl.ds(base + cc * CH, CH)], sbuf)
      pltpu.sync_copy(ew.at[c, pl.ds(base + cc * CH, CH)], ebuf)

      def blk(j, _):
        pltpu.sync_copy(table.at[gbuf.at[j]], rowbuf)
        ones = jnp.ones((16,), jnp.float32)
        for g in range(BLK // 16):
          ew16 = ebuf[j, pl.ds(16 * g, 16)]
          for l in range(16):
            w = ew16[l] * ones
            i = 16 * g + l
            for k in range(8):
              rowbuf[i, pl.ds(16 * k, 16)] = rowbuf[i, pl.ds(16 * k, 16)] * w
        pltpu.sync_copy(rowbuf, acc_sp.at[sbuf.at[j]], add=True)
        return 0

      lax.fori_loop(0, CH, blk, 0)
      return 0

    lax.fori_loop(0, nchunks, chunk, 0)
    plsc.subcore_barrier()
    pltpu.sync_copy(
        acc_sp.at[pl.ds(s * ROWS_PER_TILE, ROWS_PER_TILE)],
        out.at[pl.ds(c * NP + s * ROWS_PER_TILE, ROWS_PER_TILE)])

  mesh = plsc.VectorSubcoreMesh(core_axis_name="c", subcore_axis_name="s")
  return pl.kernel(
      body,
      out_type=jax.ShapeDtypeStruct((2 * NP, 128), jnp.float32),
      mesh=mesh,
      scratch_types=[
          pltpu.VMEM((CH, BLK), jnp.int32),
          pltpu.VMEM((CH, BLK), jnp.int32),
          pltpu.VMEM((CH, BLK), jnp.float32),
          pltpu.VMEM((BLK, 128), jnp.float32),
          pltpu.VMEM_SHARED((NP, 128), jnp.float32),
      ],
  )


def _dis_body(deg_ref, out_ref):
  d = deg_ref[0, :] + deg_ref[1, :] + 1.0
  r = lax.rsqrt(d)
  t = jnp.broadcast_to(r[None, :], (128, 128))
  out_ref[...] = jnp.transpose(t, (1, 0))


def _tc_dis(deg2):
  return pl.pallas_call(
      _dis_body,
      grid=(NP // 128,),
      in_specs=[pl.BlockSpec((2, 128), lambda m: (0, m))],
      out_specs=pl.BlockSpec((128, 128), lambda m: (m, 0)),
      out_shape=jax.ShapeDtypeStruct((NP, 128), jnp.float32),
  )(deg2)


def _mma_body(lhs_ref, w_ref, dis_ref, out_ref):
  zl = dis_ref[...] * lhs_ref[...]
  out_ref[0] = jnp.dot(zl, w_ref[...], preferred_element_type=jnp.float32)


def _tc_mm_a(lhs, w, dis2d):
  return pl.pallas_call(
      _mma_body,
      grid=(2, NP // 256),
      in_specs=[
          pl.BlockSpec((256, 128), lambda c, m: (m, 0)),
          pl.BlockSpec((128, 128), lambda c, m: (0, c)),
          pl.BlockSpec((256, 128), lambda c, m: (m, 0)),
      ],
      out_specs=pl.BlockSpec((1, 256, 128), lambda c, m: (c, m, 0)),
      out_shape=jax.ShapeDtypeStruct((2, NP, 128), jnp.float32),
  )(lhs, w, dis2d).reshape(2 * NP, 128)


def _mmb_body(l0_ref, l1_ref, w_ref, dis_ref, out_ref):
  d = dis_ref[...]
  h = jnp.dot(d * l0_ref[...], w_ref[:128, :],
              preferred_element_type=jnp.float32)
  h = h + jnp.dot(d * l1_ref[...], w_ref[128:, :],
                  preferred_element_type=jnp.float32)
  out_ref[...] = h


def _tc_mm_b(z_stacked, w, dis2d):
  nb = NP // 256
  return pl.pallas_call(
      _mmb_body,
      grid=(nb,),
      in_specs=[
          pl.BlockSpec((256, 128), lambda m: (m, 0)),
          pl.BlockSpec((256, 128), lambda m: (m + nb, 0)),
          pl.BlockSpec((256, 128), lambda m: (0, 0)),
          pl.BlockSpec((256, 128), lambda m: (m, 0)),
      ],
      out_specs=pl.BlockSpec((256, 128), lambda m: (m, 0)),
      out_shape=jax.ShapeDtypeStruct((NP, 128), jnp.float32),
  )(z_stacked, z_stacked, w, dis2d)


def _zstack_body(acc_ref, hp_ref, dis_ref, b_ref, out_ref):
  z = dis_ref[...] * (acc_ref[...] + hp_ref[...]) + b_ref[0]
  out_ref[...] = jnp.maximum(z, 0.0)


def _tc_zprep_stacked(acc, hp, dis2d, bias2):
  nb = NP // 256
  return pl.pallas_call(
      _zstack_body,
      grid=(2 * nb,),
      in_specs=[
          pl.BlockSpec((256, 128), lambda m: (m, 0)),
          pl.BlockSpec((256, 128), lambda m: (m, 0)),
          pl.BlockSpec((256, 128), lambda m: (m % nb, 0)),
          pl.BlockSpec((1, 1, 128), lambda m: (m // nb, 0, 0)),
      ],
      out_specs=pl.BlockSpec((256, 128), lambda m: (m, 0)),
      out_shape=jax.ShapeDtypeStruct((2 * NP, 128), jnp.float32),
  )(acc, hp, dis2d, bias2)


def _zsum_body(a0_ref, a1_ref, hp_ref, dis_ref, b_ref, out_ref):
  out_ref[...] = (dis_ref[...] * (a0_ref[...] + a1_ref[...] + hp_ref[...])
                  + b_ref[...])


def _tc_zprep_sum(acc_parts, hp, dis2d, bias1):
  nb = NP // 256
  return pl.pallas_call(
      _zsum_body,
      grid=(nb,),
      in_specs=[
          pl.BlockSpec((256, 128), lambda m: (m, 0)),
          pl.BlockSpec((256, 128), lambda m: (m + nb, 0)),
          pl.BlockSpec((256, 128), lambda m: (m, 0)),
          pl.BlockSpec((256, 128), lambda m: (m, 0)),
          pl.BlockSpec((1, 128), lambda m: (0, 0)),
      ],
      out_specs=pl.BlockSpec((256, 128), lambda m: (m, 0)),
      out_shape=jax.ShapeDtypeStruct((NP, 128), jnp.float32),
  )(acc_parts, acc_parts, hp, dis2d, bias1)


def kernel(x, edge_index, edge_weight, W1, b1, W2, b2, W3, b3, W4, b4):
  src = edge_index[0].astype(jnp.int32)
  dst = edge_index[1].astype(jnp.int32)
  npad = EP - E
  padv = (jnp.arange(npad, dtype=jnp.int32) % N)
  src_p = jnp.concatenate([src, padv])
  dst_p = jnp.concatenate([dst, padv])
  ew_p = jnp.concatenate(
      [edge_weight.astype(jnp.float32), jnp.zeros((npad,), jnp.float32)])

  gidx_f = jnp.stack([src_p, src_p + NP]).reshape(2, NBF, BLK)
  sidx_f = jnp.stack([dst_p, dst_p]).reshape(2, NBF, BLK)
  ew_f = jnp.stack([ew_p, ew_p]).reshape(2, NBF, BLK)
  gidx_e = src_p.reshape(2, NBE, BLK)
  sidx_e = dst_p.reshape(2, NBE, BLK)
  ew_e = ew_p.reshape(2, NBE, BLK)

  x_pad = jnp.pad(x, ((0, NP - N), (0, 0)))
  w2p = jnp.pad(W2, ((0, 0), (0, 128 - W2.shape[1])))
  w3p = jnp.pad(W3, ((0, 128 - W3.shape[0]), (0, 0)))
  bs1 = b1.reshape(2, 1, 128)
  b2p = jnp.pad(b2, (0, 128 - b2.shape[0])).reshape(1, 128)
  bs3 = b3.reshape(2, 1, 128)
  b4r = b4.reshape(1, 128)

  deg_kernel = _make_deg_kernel()
  msg_f = _make_msg_kernel(NBF, 2 * NP)
  msg_e = _make_msg_kernel(NBE, NP)

  deg2 = deg_kernel(sidx_e, ew_e)
  dis2d = _tc_dis(deg2)

  h1 = _tc_mm_a(x_pad, W1, dis2d)
  a1 = msg_f(h1, gidx_f, sidx_f, ew_f)
  z1 = _tc_zprep_stacked(a1, h1, dis2d, bs1)

  h2 = _tc_mm_b(z1, w2p, dis2d)
  a2 = msg_e(h2, gidx_e, sidx_e, ew_e)
  z2 = _tc_zprep_sum(a2, h2, dis2d, b2p)

  h3 = _tc_mm_a(z2, w3p, dis2d)
  a3 = msg_f(h3, gidx_f, sidx_f, ew_f)
  z3 = _tc_zprep_stacked(a3, h3, dis2d, bs3)

  h4 = _tc_mm_b(z3, W4, dis2d)
  a4 = msg_e(h4, gidx_e, sidx_e, ew_e)
  out = _tc_zprep_sum(a4, h4, dis2d, b4r)

  return out[:N]
